# baseline (speedup 1.0000x reference)
import functools

import jax
import jax.numpy as jnp
from jax import lax
from jax.experimental import pallas as pl
from jax.experimental.pallas import tpu as pltpu
from jax.experimental.pallas import tpu_sc as plsc

N = 10000
E = 320000
F = 128
EPS = 1e-5

NC = 2
NS = 16
NW = NC * NS
EDGES_PER_W = E // NW
CHUNK = 128
NCHUNK = 80
EPAD = NCHUNK * CHUNK - EDGES_PER_W
N_PAD = 10240
ROWS_PER_TILE = N_PAD // NS


def _mesh():
    return plsc.VectorSubcoreMesh(core_axis_name="c", subcore_axis_name="s")


@functools.partial(
    pl.kernel,
    out_type=jax.ShapeDtypeStruct((NC * N_PAD, F), jnp.float32),
    mesh=_mesh(),
    scratch_types=[
        pltpu.VMEM((NCHUNK, CHUNK), jnp.int32),
        pltpu.VMEM((CHUNK, F), jnp.float32),
        pltpu.VMEM_SHARED((N_PAD, F), jnp.float32),
    ],
)
def _sc_degree(dst_hbm, zeros_hbm, ones_hbm, out_hbm, dst_v, ones_v,
               shared_deg):
    c = lax.axis_index("c")
    s = lax.axis_index("s")
    wid = c * NS + s

    pltpu.sync_copy(
        zeros_hbm.at[pl.ds(s * ROWS_PER_TILE, ROWS_PER_TILE)],
        shared_deg.at[pl.ds(s * ROWS_PER_TILE, ROWS_PER_TILE)],
    )
    pltpu.sync_copy(ones_hbm, ones_v)
    pltpu.sync_copy(dst_hbm.at[wid], dst_v)
    plsc.subcore_barrier()

    def body(i, carry):
        pltpu.sync_copy(ones_v, shared_deg.at[dst_v.at[i]], add=True)
        return carry

    lax.fori_loop(0, NCHUNK, body, 0)
    plsc.subcore_barrier()

    pltpu.sync_copy(
        shared_deg.at[pl.ds(s * ROWS_PER_TILE, ROWS_PER_TILE)],
        out_hbm.at[pl.ds(c * N_PAD + s * ROWS_PER_TILE, ROWS_PER_TILE)],
    )


@functools.partial(
    pl.kernel,
    out_type=jax.ShapeDtypeStruct((NC * N_PAD, F), jnp.float32),
    mesh=_mesh(),
    scratch_types=[
        pltpu.VMEM((CHUNK,), jnp.int32),
        pltpu.VMEM((CHUNK,), jnp.int32),
        pltpu.VMEM((NCHUNK, CHUNK), jnp.int32),
        pltpu.VMEM((CHUNK, F), jnp.float32),
        pltpu.VMEM((CHUNK, F), jnp.float32),
        pltpu.VMEM_SHARED((N_PAD, F), jnp.float32),
        pltpu.SemaphoreType.DMA,
        pltpu.SemaphoreType.DMA,
        pltpu.SemaphoreType.DMA,
        pltpu.SemaphoreType.DMA,
        pltpu.SemaphoreType.DMA,
        pltpu.SemaphoreType.DMA,
    ],
)
def _sc_agg(h_hbm, src_hbm, dst_hbm, zeros_hbm, out_hbm,
            idx_a, idx_b, dst_v, rows_a, rows_b, shared_agg,
            gsem_a, gsem_b, ssem_a, ssem_b, isem_a, isem_b):
    c = lax.axis_index("c")
    s = lax.axis_index("s")
    wid = c * NS + s

    pltpu.sync_copy(
        zeros_hbm.at[pl.ds(s * ROWS_PER_TILE, ROWS_PER_TILE)],
        shared_agg.at[pl.ds(s * ROWS_PER_TILE, ROWS_PER_TILE)],
    )
    pltpu.sync_copy(dst_hbm.at[wid], dst_v)
    pltpu.async_copy(src_hbm.at[wid, 0], idx_a, isem_a)
    pltpu.async_copy(src_hbm.at[wid, 1], idx_b, isem_b)
    pltpu.make_async_copy(src_hbm.at[wid, 0], idx_a, isem_a).wait()
    pltpu.async_copy(h_hbm.at[idx_a], rows_a, gsem_a)
    plsc.subcore_barrier()

    def step(i, idx_cur, isem_cur, rows_cur, gsem_cur, ssem_cur,
             idx_nxt, isem_nxt, rows_nxt, gsem_nxt, ssem_nxt):
        pltpu.make_async_copy(h_hbm.at[idx_cur], rows_cur, gsem_cur).wait()

        @pl.when(i >= 1)
        def _():
            pltpu.make_async_copy(rows_nxt, shared_agg.at[dst_v.at[i - 1]],
                                  ssem_nxt).wait()

        pltpu.make_async_copy(src_hbm.at[wid, i + 1], idx_nxt,
                              isem_nxt).wait()
        pltpu.async_copy(src_hbm.at[wid, i + 2], idx_cur, isem_cur)
        pltpu.async_copy(h_hbm.at[idx_nxt], rows_nxt, gsem_nxt)
        pltpu.async_copy(rows_cur, shared_agg.at[dst_v.at[i]], ssem_cur,
                         add=True)

    def body(i, carry):
        @pl.when(i % 2 == 0)
        def _():
            step(i, idx_a, isem_a, rows_a, gsem_a, ssem_a,
                 idx_b, isem_b, rows_b, gsem_b, ssem_b)

        @pl.when(i % 2 == 1)
        def _():
            step(i, idx_b, isem_b, rows_b, gsem_b, ssem_b,
                 idx_a, isem_a, rows_a, gsem_a, ssem_a)

        return carry

    lax.fori_loop(0, NCHUNK, body, 0)
    pltpu.make_async_copy(rows_b, shared_agg.at[dst_v.at[NCHUNK - 1]],
                          ssem_b).wait()
    pltpu.make_async_copy(h_hbm.at[idx_a], rows_a, gsem_a).wait()
    pltpu.make_async_copy(src_hbm.at[wid, NCHUNK + 1], idx_b, isem_b).wait()
    plsc.subcore_barrier()

    pltpu.sync_copy(
        shared_agg.at[pl.ds(s * ROWS_PER_TILE, ROWS_PER_TILE)],
        out_hbm.at[pl.ds(c * N_PAD + s * ROWS_PER_TILE, ROWS_PER_TILE)],
    )


def _tc_dense_bn(h, agg2, deg2, w_self, w_neigh, b, gamma, beta):
    def body(h_ref, agg_ref, deg_ref, ws_ref, wn_ref, b_ref, g_ref, be_ref,
             o_ref):
        deg = deg_ref[0:N, :] + deg_ref[N_PAD:N_PAD + N, :]
        agg = agg_ref[0:N, :] + agg_ref[N_PAD:N_PAD + N, :]
        hn = agg / jnp.maximum(deg, 1.0)
        z = (jnp.dot(h_ref[...], ws_ref[...], preferred_element_type=jnp.float32)
             + jnp.dot(hn, wn_ref[...], preferred_element_type=jnp.float32)
             + b_ref[...])
        mu = jnp.mean(z, axis=0, keepdims=True)
        var = jnp.mean((z - mu) ** 2, axis=0, keepdims=True)
        z = (z - mu) * lax.rsqrt(var + EPS) * g_ref[...] + be_ref[...]
        o_ref[...] = jnp.maximum(z, 0.0)

    return pl.pallas_call(
        body,
        out_shape=jax.ShapeDtypeStruct((N, F), jnp.float32),
    )(h, agg2, deg2, w_self, w_neigh, b.reshape(1, F), gamma.reshape(1, F),
      beta.reshape(1, F))


def _tc_dense(h, agg2, deg2, w_self, w_neigh, b):
    def body(h_ref, agg_ref, deg_ref, ws_ref, wn_ref, b_ref, o_ref):
        deg = deg_ref[0:N, :] + deg_ref[N_PAD:N_PAD + N, :]
        agg = agg_ref[0:N, :] + agg_ref[N_PAD:N_PAD + N, :]
        hn = agg / jnp.maximum(deg, 1.0)
        o_ref[...] = (
            jnp.dot(h_ref[...], ws_ref[...], preferred_element_type=jnp.float32)
            + jnp.dot(hn, wn_ref[...], preferred_element_type=jnp.float32)
            + b_ref[...])

    return pl.pallas_call(
        body,
        out_shape=jax.ShapeDtypeStruct((N, F), jnp.float32),
    )(h, agg2, deg2, w_self, w_neigh, b.reshape(1, F))


def kernel(x, edge_index, W_self1, W_neigh1, b1, gamma1, beta1,
           W_self2, W_neigh2, b2, gamma2, beta2,
           W_self3, W_neigh3, b3):
    src4 = jnp.pad(edge_index[0].reshape(NW, EDGES_PER_W),
                   ((0, 0), (0, EPAD))).reshape(NW, NCHUNK, CHUNK)
    src4 = jnp.concatenate([src4, src4[:, :2, :]], axis=1)
    dst4 = jnp.pad(edge_index[1].reshape(NW, EDGES_PER_W),
                   ((0, 0), (0, EPAD)),
                   constant_values=N_PAD - 1).reshape(NW, NCHUNK, CHUNK)
    zeros_nf = jnp.zeros((N_PAD, F), jnp.float32)
    ones_cf = jnp.ones((CHUNK, F), jnp.float32)

    deg2 = _sc_degree(dst4, zeros_nf, ones_cf)[:, 0:1]
    agg2 = _sc_agg(x, src4, dst4, zeros_nf)
    h = _tc_dense_bn(x, agg2, deg2, W_self1, W_neigh1, b1, gamma1, beta1)
    agg2 = _sc_agg(h, src4, dst4, zeros_nf)
    h = _tc_dense_bn(h, agg2, deg2, W_self2, W_neigh2, b2, gamma2, beta2)
    agg2 = _sc_agg(h, src4, dst4, zeros_nf)
    h = _tc_dense(h, agg2, deg2, W_self3, W_neigh3, b3)
    return h


# speedup vs baseline: 2.2500x; 2.2500x over previous
import functools

import jax
import jax.numpy as jnp
from jax import lax
from jax.experimental import pallas as pl
from jax.experimental.pallas import tpu as pltpu
from jax.experimental.pallas import tpu_sc as plsc

N = 10000
E = 320000
F = 128
EPS = 1e-5

NC = 2
NS = 16
NW = NC * NS
EDGES_PER_W = E // NW
CHUNK = 80
NCHUNK = EDGES_PER_W // CHUNK
N_PAD = 10240
ROWS_PER_TILE = N_PAD // NS


def _mesh():
    return plsc.VectorSubcoreMesh(core_axis_name="c", subcore_axis_name="s")


@functools.partial(
    pl.kernel,
    out_type=jax.ShapeDtypeStruct((NC * N_PAD, F), jnp.float32),
    mesh=_mesh(),
    scratch_types=[
        pltpu.VMEM((NCHUNK, CHUNK), jnp.int32),
        pltpu.VMEM((CHUNK, F), jnp.float32),
        pltpu.VMEM_SHARED((N_PAD, F), jnp.float32),
    ],
)
def _sc_degree(dst_hbm, zeros_hbm, ones_hbm, out_hbm, dst_v, ones_v,
               shared_deg):
    c = lax.axis_index("c")
    s = lax.axis_index("s")
    wid = c * NS + s

    pltpu.sync_copy(
        zeros_hbm.at[pl.ds(s * ROWS_PER_TILE, ROWS_PER_TILE)],
        shared_deg.at[pl.ds(s * ROWS_PER_TILE, ROWS_PER_TILE)],
    )
    pltpu.sync_copy(ones_hbm, ones_v)
    pltpu.sync_copy(dst_hbm.at[wid], dst_v)
    plsc.subcore_barrier()

    def body(i, carry):
        pltpu.sync_copy(ones_v, shared_deg.at[dst_v.at[i]], add=True)
        return carry

    lax.fori_loop(0, NCHUNK, body, 0)
    plsc.subcore_barrier()

    pltpu.sync_copy(
        shared_deg.at[pl.ds(s * ROWS_PER_TILE, ROWS_PER_TILE)],
        out_hbm.at[pl.ds(c * N_PAD + s * ROWS_PER_TILE, ROWS_PER_TILE)],
    )


@functools.partial(
    pl.kernel,
    out_type=jax.ShapeDtypeStruct((NC * N_PAD, F), jnp.float32),
    mesh=_mesh(),
    scratch_types=[
        pltpu.VMEM(((NCHUNK + 1) * CHUNK,), jnp.int32),
        pltpu.VMEM((NCHUNK, CHUNK), jnp.int32),
        pltpu.VMEM((CHUNK, F), jnp.float32),
        pltpu.VMEM((CHUNK, F), jnp.float32),
        pltpu.VMEM_SHARED((N_PAD, F), jnp.float32),
        pltpu.SemaphoreType.DMA,
        pltpu.SemaphoreType.DMA,
        pltpu.SemaphoreType.DMA,
        pltpu.SemaphoreType.DMA,
    ],
)
def _sc_agg(h_hbm, src_hbm, dst_hbm, zeros_hbm, out_hbm,
            src_v, dst_v, rows_a, rows_b, shared_agg, sem_a, sem_b,
            ssem_a, ssem_b):
    c = lax.axis_index("c")
    s = lax.axis_index("s")
    wid = c * NS + s

    pltpu.sync_copy(
        zeros_hbm.at[pl.ds(s * ROWS_PER_TILE, ROWS_PER_TILE)],
        shared_agg.at[pl.ds(s * ROWS_PER_TILE, ROWS_PER_TILE)],
    )
    pltpu.sync_copy(src_hbm.at[pl.ds(wid * (NCHUNK + 1) * CHUNK,
                                     (NCHUNK + 1) * CHUNK)], src_v)
    pltpu.sync_copy(dst_hbm.at[wid], dst_v)
    plsc.subcore_barrier()

    pltpu.async_copy(h_hbm.at[src_v.at[pl.ds(0, CHUNK)]], rows_a, sem_a)

    def step(i, rows_cur, sem_cur, ssem_cur, rows_nxt, sem_nxt, ssem_nxt):
        pltpu.make_async_copy(h_hbm.at[src_v.at[pl.ds(i * CHUNK, CHUNK)]],
                              rows_cur, sem_cur).wait()

        @pl.when(i >= 1)
        def _():
            pltpu.make_async_copy(rows_nxt, shared_agg.at[dst_v.at[i - 1]],
                                  ssem_nxt).wait()

        pltpu.async_copy(h_hbm.at[src_v.at[pl.ds((i + 1) * CHUNK, CHUNK)]],
                         rows_nxt, sem_nxt, priority=1)
        pltpu.async_copy(rows_cur, shared_agg.at[dst_v.at[i]], ssem_cur,
                         add=True)

    def body(i, carry):
        @pl.when(i % 2 == 0)
        def _():
            step(i, rows_a, sem_a, ssem_a, rows_b, sem_b, ssem_b)

        @pl.when(i % 2 == 1)
        def _():
            step(i, rows_b, sem_b, ssem_b, rows_a, sem_a, ssem_a)

        return carry

    lax.fori_loop(0, NCHUNK, body, 0)
    rows_l, ssem_l = (rows_a, ssem_a) if (NCHUNK - 1) % 2 == 0 else (rows_b, ssem_b)
    pltpu.make_async_copy(rows_l, shared_agg.at[dst_v.at[NCHUNK - 1]],
                          ssem_l).wait()
    rows_d, sem_d = (rows_a, sem_a) if NCHUNK % 2 == 0 else (rows_b, sem_b)
    pltpu.make_async_copy(h_hbm.at[src_v.at[pl.ds(NCHUNK * CHUNK, CHUNK)]],
                          rows_d, sem_d).wait()
    plsc.subcore_barrier()

    pltpu.sync_copy(
        shared_agg.at[pl.ds(s * ROWS_PER_TILE, ROWS_PER_TILE)],
        out_hbm.at[pl.ds(c * N_PAD + s * ROWS_PER_TILE, ROWS_PER_TILE)],
    )


def _tc_dense_bn(h, agg2, deg2, w_self, w_neigh, b, gamma, beta):
    def body(h_ref, agg_ref, deg_ref, ws_ref, wn_ref, b_ref, g_ref, be_ref,
             o_ref):
        deg = deg_ref[0:N, :] + deg_ref[N_PAD:N_PAD + N, :]
        agg = agg_ref[0:N, :] + agg_ref[N_PAD:N_PAD + N, :]
        hn = agg / jnp.maximum(deg, 1.0)
        z = (jnp.dot(h_ref[...], ws_ref[...], preferred_element_type=jnp.float32)
             + jnp.dot(hn, wn_ref[...], preferred_element_type=jnp.float32)
             + b_ref[...])
        mu = jnp.mean(z, axis=0, keepdims=True)
        var = jnp.mean((z - mu) ** 2, axis=0, keepdims=True)
        z = (z - mu) * lax.rsqrt(var + EPS) * g_ref[...] + be_ref[...]
        o_ref[...] = jnp.maximum(z, 0.0)

    return pl.pallas_call(
        body,
        out_shape=jax.ShapeDtypeStruct((N, F), jnp.float32),
    )(h, agg2, deg2, w_self, w_neigh, b.reshape(1, F), gamma.reshape(1, F),
      beta.reshape(1, F))


def _tc_dense(h, agg2, deg2, w_self, w_neigh, b):
    def body(h_ref, agg_ref, deg_ref, ws_ref, wn_ref, b_ref, o_ref):
        deg = deg_ref[0:N, :] + deg_ref[N_PAD:N_PAD + N, :]
        agg = agg_ref[0:N, :] + agg_ref[N_PAD:N_PAD + N, :]
        hn = agg / jnp.maximum(deg, 1.0)
        o_ref[...] = (
            jnp.dot(h_ref[...], ws_ref[...], preferred_element_type=jnp.float32)
            + jnp.dot(hn, wn_ref[...], preferred_element_type=jnp.float32)
            + b_ref[...])

    return pl.pallas_call(
        body,
        out_shape=jax.ShapeDtypeStruct((N, F), jnp.float32),
    )(h, agg2, deg2, w_self, w_neigh, b.reshape(1, F))


def kernel(x, edge_index, W_self1, W_neigh1, b1, gamma1, beta1,
           W_self2, W_neigh2, b2, gamma2, beta2,
           W_self3, W_neigh3, b3):
    src3 = edge_index[0].reshape(NW, NCHUNK, CHUNK)
    src3 = jnp.concatenate([src3, src3[:, :1, :]], axis=1).reshape(-1)
    dst3 = edge_index[1].reshape(NW, NCHUNK, CHUNK)
    zeros_nf = jnp.zeros((N_PAD, F), jnp.float32)
    ones_cf = jnp.ones((CHUNK, F), jnp.float32)

    deg2 = _sc_degree(dst3, zeros_nf, ones_cf)[:, 0:1]
    agg2 = _sc_agg(x, src3, dst3, zeros_nf)
    h = _tc_dense_bn(x, agg2, deg2, W_self1, W_neigh1, b1, gamma1, beta1)
    agg2 = _sc_agg(h, src3, dst3, zeros_nf)
    h = _tc_dense_bn(h, agg2, deg2, W_self2, W_neigh2, b2, gamma2, beta2)
    agg2 = _sc_agg(h, src3, dst3, zeros_nf)
    h = _tc_dense(h, agg2, deg2, W_self3, W_neigh3, b3)
    return h
